# Initial kernel scaffold; baseline (speedup 1.0000x reference)
import jax
import jax.numpy as jnp
from jax.experimental import pallas as pl


def kernel(x, edge_index, W1, b1, W2, b2, gn1_w, gn1_b, gn1_a, gn2_w, gn2_b, gn2_a, Wr, br):
    raise NotImplementedError("write your pallas kernel here")


# speedup vs baseline: 13.0884x; 13.0884x over previous
import functools

import jax
import jax.numpy as jnp
from jax import lax
from jax.experimental import pallas as pl
from jax.experimental.pallas import tpu as pltpu
from jax.experimental.pallas import tpu_sc as plsc

_EPS = 1e-5
_NC = 2
_NS = 16
_NW = _NC * _NS
_K = 80


def _deg_sc(dst, n):
    e = dst.shape[0]
    epw = e // _NW
    ch = epw // _K
    npad = -(-n // 128) * 128
    mesh = plsc.VectorSubcoreMesh(core_axis_name="c", subcore_axis_name="s")

    @functools.partial(
        pl.kernel,
        out_type=jax.ShapeDtypeStruct((_NC * npad,), jnp.float32),
        mesh=mesh,
        scratch_types=[
            pltpu.VMEM((_K,), jnp.int32),
            pltpu.VMEM((_K,), jnp.float32),
            pltpu.VMEM((npad,), jnp.float32),
            pltpu.VMEM_SHARED((npad,), jnp.float32),
        ],
    )
    def body(dst_hbm, out_hbm, didx, ones, zbuf, acc):
        cid = lax.axis_index("c")
        sid = lax.axis_index("s")
        wid = sid * _NC + cid
        for j in range(_K // 16):
            ones[pl.ds(j * 16, 16)] = jnp.full((16,), 1.0, jnp.float32)

        @pl.when(sid == 0)
        def _zero():
            def zstep(i, c):
                zbuf[pl.ds(i * 16, 16)] = jnp.zeros((16,), jnp.float32)
                return c
            lax.fori_loop(0, npad // 16, zstep, 0)
            pltpu.sync_copy(zbuf, acc)

        plsc.subcore_barrier()

        def step(i, c):
            base = wid * epw + i * _K
            pltpu.sync_copy(dst_hbm.at[pl.ds(base, _K)], didx)
            pltpu.sync_copy(ones, acc.at[didx], add=True)
            return c

        lax.fori_loop(0, ch, step, 0)
        plsc.subcore_barrier()

        @pl.when(sid == 0)
        def _out():
            pltpu.sync_copy(acc, out_hbm.at[pl.ds(cid * npad, npad)])

    return body(dst)


def _seg_sum_sc(g, src, dst, zeros):
    n, d = g.shape
    e = src.shape[0]
    epw = e // _NW
    ch = epw // _K
    rpt = (n // _NS) // 8 * 8
    tail = n - _NS * rpt
    mesh = plsc.VectorSubcoreMesh(core_axis_name="c", subcore_axis_name="s")

    @functools.partial(
        pl.kernel,
        out_type=jax.ShapeDtypeStruct((_NC, n, d), jnp.float32),
        mesh=mesh,
        scratch_types=[
            pltpu.VMEM((_K,), jnp.int32),
            pltpu.VMEM((_K,), jnp.int32),
            pltpu.VMEM((_K, d), jnp.float32),
            pltpu.VMEM_SHARED((n, d), jnp.float32),
            pltpu.SemaphoreType.DMA,
        ],
    )
    def body(g_hbm, src_hbm, dst_hbm, z_hbm, out_hbm, sidx, didx, rows, acc, sem):
        cid = lax.axis_index("c")
        sid = lax.axis_index("s")
        wid = sid * _NC + cid
        pltpu.sync_copy(z_hbm.at[pl.ds(sid * rpt, rpt)],
                        acc.at[pl.ds(sid * rpt, rpt)])
        if tail:
            @pl.when(sid == 0)
            def _ztail():
                pltpu.sync_copy(z_hbm.at[pl.ds(_NS * rpt, tail)],
                                acc.at[pl.ds(_NS * rpt, tail)])
        plsc.subcore_barrier()

        def step(i, c):
            base = wid * epw + i * _K
            pltpu.sync_copy(src_hbm.at[pl.ds(base, _K)], sidx)
            pltpu.sync_copy(dst_hbm.at[pl.ds(base, _K)], didx)
            pltpu.async_copy(g_hbm.at[sidx], rows, sem).wait()
            pltpu.sync_copy(rows, acc.at[didx], add=True)
            return c

        lax.fori_loop(0, ch, step, 0)
        plsc.subcore_barrier()
        pltpu.sync_copy(acc.at[pl.ds(sid * rpt, rpt)],
                        out_hbm.at[cid, pl.ds(sid * rpt, rpt)])
        if tail:
            @pl.when(sid == 0)
            def _otail():
                pltpu.sync_copy(acc.at[pl.ds(_NS * rpt, tail)],
                                out_hbm.at[cid, pl.ds(_NS * rpt, tail)])

    return body(g, src, dst, zeros)


def _tc1(x, w1, deg_t):
    n, d = x.shape

    def body(x_ref, w_ref, deg_ref, g1_ref, dinv_ref):
        deg = deg_ref[:, 0:1] + deg_ref[:, 1:2] + 1.0
        dinv = lax.rsqrt(deg)
        h = jnp.dot(x_ref[...], w_ref[...], preferred_element_type=jnp.float32)
        g1_ref[...] = h * dinv
        dinv_ref[...] = dinv

    return pl.pallas_call(
        body,
        out_shape=(jax.ShapeDtypeStruct((n, d), jnp.float32),
                   jax.ShapeDtypeStruct((n, 1), jnp.float32)),
    )(x, w1, deg_t)


def _tc2(s1p, g1, dinv, b1, gnw, gnb, gna, w2):
    n, d = g1.shape

    def body(sp_ref, g_ref, di_ref, b_ref, w_ref, bt_ref, a_ref, w2_ref,
             x1_ref, g2_ref):
        s = sp_ref[0] + sp_ref[1] + g_ref[...]
        y = di_ref[...] * s + b_ref[...]
        mean = jnp.mean(y, axis=0, keepdims=True)
        o = y - a_ref[...] * mean
        var = jnp.mean(o * o, axis=0, keepdims=True)
        x1 = jnp.maximum(w_ref[...] * o * lax.rsqrt(var + _EPS) + bt_ref[...],
                         0.0)
        x1_ref[...] = x1
        g2_ref[...] = jnp.dot(x1, w2_ref[...],
                              preferred_element_type=jnp.float32) * di_ref[...]

    return pl.pallas_call(
        body,
        out_shape=(jax.ShapeDtypeStruct((n, d), jnp.float32),
                   jax.ShapeDtypeStruct((n, d), jnp.float32)),
    )(s1p, g1, dinv, b1, gnw, gnb, gna, w2)


def _tc3(s2p, g2, dinv, b2, gnw, gnb, gna, x1, wr, br):
    n, d = g2.shape

    def body(sp_ref, g_ref, di_ref, b_ref, w_ref, bt_ref, a_ref, x1_ref,
             wr_ref, br_ref, out_ref):
        s = sp_ref[0] + sp_ref[1] + g_ref[...]
        y = di_ref[...] * s + b_ref[...]
        mean = jnp.mean(y, axis=0, keepdims=True)
        o = y - a_ref[...] * mean
        var = jnp.mean(o * o, axis=0, keepdims=True)
        x2 = jnp.maximum(w_ref[...] * o * lax.rsqrt(var + _EPS) + bt_ref[...],
                         0.0)
        out_ref[...] = jnp.dot(x1_ref[...] + x2, wr_ref[...],
                               preferred_element_type=jnp.float32) + br_ref[...]

    return pl.pallas_call(
        body,
        out_shape=jax.ShapeDtypeStruct((n, d), jnp.float32),
    )(s2p, g2, dinv, b2, gnw, gnb, gna, x1, wr, br)


def kernel(x, edge_index, W1, b1, W2, b2, gn1_w, gn1_b, gn1_a, gn2_w, gn2_b,
           gn2_a, Wr, br):
    n, d = x.shape
    e = edge_index.shape[1]
    assert e % (_NW * _K) == 0 and n % _NS == 0 and n % 16 == 0

    src = edge_index[0]
    dst = edge_index[1]
    zeros = jnp.zeros((n, d), jnp.float32)

    npad = -(-n // 128) * 128
    degp = _deg_sc(dst, n).reshape(_NC, npad)[:, :n]
    deg_t = degp.T

    g1, dinv = _tc1(x, W1, deg_t)
    s1p = _seg_sum_sc(g1, src, dst, zeros)
    x1, g2 = _tc2(s1p, g1, dinv, b1.reshape(1, d), gn1_w.reshape(1, d),
                  gn1_b.reshape(1, d), gn1_a.reshape(1, d), W2)
    s2p = _seg_sum_sc(g2, src, dst, zeros)
    return _tc3(s2p, g2, dinv, b2.reshape(1, d), gn2_w.reshape(1, d),
                gn2_b.reshape(1, d), gn2_a.reshape(1, d), x1, Wr, br.reshape(1, d))
